# Initial kernel scaffold; baseline (speedup 1.0000x reference)
import jax
import jax.numpy as jnp
from jax.experimental import pallas as pl


def kernel(target, context, emb_table, softmax_w_table, softmax_b_table):
    raise NotImplementedError("write your pallas kernel here")


# speedup vs baseline: 8.6026x; 8.6026x over previous
import functools

import jax
import jax.numpy as jnp
from jax import lax
from jax.experimental import pallas as pl
from jax.experimental.pallas import tpu as pltpu
from jax.experimental.pallas import tpu_sc as plsc

D = 64
NEGS = 64
S = 1 + NEGS
SP = 72
L = 16
NC = 2
NSUB = 16
NW = NC * NSUB
CH = 128
NBUF = 4

NEG_BIG = -1e30


def _splat(x):
    return jnp.full((L,), x, dtype=jnp.int32)


def _build_sc_call(B):
    RPW = B // NW
    NCHUNK = RPW // CH
    mesh = plsc.VectorSubcoreMesh(
        core_axis_name="c", subcore_axis_name="s",
        num_cores=NC, num_subcores=NSUB)

    def body(samples_hbm, ctx_hbm, emb_hbm, w_hbm, b_hbm, out_hbm,
             samples_v, ctx_v, e_v, w_v, b_v, logits_v, out_v,
             sem_in, sem_w, sem_b):
        wid = lax.axis_index("s") * NC + lax.axis_index("c")
        lanes = lax.iota(jnp.int32, L)
        last_lane = lanes == (L - 1)

        def w_idx_ref(row):
            off = pl.multiple_of(row * SP, 8)
            return samples_v.at[pl.ds(off, SP)]

        def b_dst(slot):
            return b_v.at[pl.ds(slot * 80, SP)]

        def start_row(row, slot):
            idx = w_idx_ref(row)
            pltpu.make_async_copy(w_hbm.at[idx], w_v.at[slot],
                                  sem_w.at[slot]).start()
            pltpu.make_async_copy(b_hbm.at[idx], b_dst(slot),
                                  sem_b.at[slot]).start()

        def wait_row(row, slot):
            idx = w_idx_ref(row)
            pltpu.make_async_copy(w_hbm.at[idx], w_v.at[slot],
                                  sem_w.at[slot]).wait()
            pltpu.make_async_copy(b_hbm.at[idx], b_dst(slot),
                                  sem_b.at[slot]).wait()

        def compute_row(row, slot):
            e = [e_v[row, pl.ds(k * L, L)] for k in range(D // L)]

            def s_group(sg, _):
                for u in range(5):
                    s = sg * 5 + u
                    s_spl = _splat(s)
                    acc = e[0] * w_v[slot, s, pl.ds(0, L)]
                    for k in range(1, D // L):
                        acc = acc + e[k] * w_v[slot, s, pl.ds(k * L, L)]
                    tot = plsc.cumsum(acc)
                    plsc.store_scatter(logits_v, [s_spl], tot,
                                       mask=last_lane)
                return 0

            lax.fori_loop(0, S // 5, s_group, 0)

            lvs = []
            for k in range(5):
                raw = (logits_v[pl.ds(k * L, L)]
                       + b_v[pl.ds(slot * 80 + k * L, L)])
                if (k + 1) * L <= S:
                    lvs.append(raw)
                else:
                    valid = (lanes + k * L) < S
                    lvs.append(jnp.where(valid, raw, NEG_BIG))
            mx = jnp.maximum(jnp.maximum(lvs[0], lvs[1]),
                             jnp.maximum(lvs[2], lvs[3]))
            mx = jnp.maximum(mx, lvs[4])
            mb = jnp.full((L,), jnp.max(mx))
            exs = [jnp.exp(v - mb) for v in lvs]
            sb = jnp.full((L,), jnp.sum(exs[0] + exs[1] + exs[2]
                                        + exs[3] + exs[4]))
            base = row * SP
            for k in range(5):
                idx = _splat(base + k * L) + lanes
                if (k + 1) * L <= SP:
                    plsc.store_scatter(out_v, [idx], exs[k] / sb)
                else:
                    valid = (lanes + k * L) < SP
                    plsc.store_scatter(out_v, [idx], exs[k] / sb,
                                       mask=valid)

        def chunk_body(c, _):
            base_row = wid * RPW + c * CH
            off = pl.multiple_of(base_row * SP, 8)
            pltpu.sync_copy(samples_hbm.at[pl.ds(off, CH * SP)], samples_v)
            pltpu.sync_copy(
                ctx_hbm.at[pl.ds(pl.multiple_of(base_row, 8), CH)], ctx_v)
            pltpu.make_async_copy(emb_hbm.at[ctx_v], e_v, sem_in).start()
            pltpu.make_async_copy(emb_hbm.at[ctx_v], e_v, sem_in).wait()

            for j in range(NBUF):
                start_row(j, j)

            def rg_body(rg, _):
                for j in range(NBUF):
                    row = rg * NBUF + j
                    wait_row(row, j)
                    compute_row(row, j)
                    nxt = row + NBUF

                    @pl.when(nxt < CH)
                    def _issue():
                        start_row(nxt, j)
                return 0

            lax.fori_loop(0, CH // NBUF, rg_body, 0)
            pltpu.sync_copy(out_v, out_hbm.at[pl.ds(off, CH * SP)])
            return 0

        lax.fori_loop(0, NCHUNK, chunk_body, 0)

    return pl.kernel(
        body,
        out_type=jax.ShapeDtypeStruct((B * SP,), jnp.float32),
        mesh=mesh,
        compiler_params=pltpu.CompilerParams(
            needs_layout_passes=False, use_tc_tiling_on_sc=False),
        scratch_types=[
            pltpu.VMEM((CH * SP,), jnp.int32),
            pltpu.VMEM((CH,), jnp.int32),
            pltpu.VMEM((CH, D), jnp.float32),
            pltpu.VMEM((NBUF, SP, D), jnp.float32),
            pltpu.VMEM((NBUF * 80,), jnp.float32),
            pltpu.VMEM((80,), jnp.float32),
            pltpu.VMEM((CH * SP,), jnp.float32),
            pltpu.SemaphoreType.DMA,
            pltpu.SemaphoreType.DMA((NBUF,)),
            pltpu.SemaphoreType.DMA((NBUF,)),
        ],
    )


def kernel(target, context, emb_table, softmax_w_table, softmax_b_table):
    B = target.shape[0]
    V = emb_table.shape[0]
    negatives = jax.random.randint(
        jax.random.key(42), (B, NEGS), 0, V, dtype=jnp.int32)
    samples = jnp.concatenate([target, negatives], axis=1)
    samples = jnp.pad(samples, ((0, 0), (0, SP - S)))
    out_flat = _build_sc_call(B)(
        samples.reshape(B * SP),
        context.reshape(B),
        emb_table,
        softmax_w_table,
        softmax_b_table.reshape(V),
    )
    return out_flat.reshape(B, SP)[:, :S]
